# Initial kernel scaffold; baseline (speedup 1.0000x reference)
import jax
import jax.numpy as jnp
from jax.experimental import pallas as pl


def kernel(x, pe):
    raise NotImplementedError("write your pallas kernel here")


# speedup vs baseline: 1.9633x; 1.9633x over previous
import jax
import jax.numpy as jnp
from jax.experimental import pallas as pl


_S_BLK = 512


def _add_body(x_ref, pe_ref, o_ref):
    o_ref[...] = x_ref[...] + pe_ref[...][None, :, :]


def kernel(x, pe):
    B, S, D = x.shape
    grid = (S // _S_BLK,)
    return pl.pallas_call(
        _add_body,
        grid=grid,
        in_specs=[
            pl.BlockSpec((B, _S_BLK, D), lambda i: (0, i, 0)),
            pl.BlockSpec((_S_BLK, D), lambda i: (i, 0)),
        ],
        out_specs=pl.BlockSpec((B, _S_BLK, D), lambda i: (0, i, 0)),
        out_shape=jax.ShapeDtypeStruct((B, S, D), x.dtype),
    )(x, pe)
